# Initial kernel scaffold; baseline (speedup 1.0000x reference)
import jax
import jax.numpy as jnp
from jax.experimental import pallas as pl


def kernel(input_ids, hidden_states, positions, embed_table):
    raise NotImplementedError("write your pallas kernel here")


# speedup vs baseline: 1.0127x; 1.0127x over previous
import functools

import jax
import jax.numpy as jnp
from jax import lax
from jax.experimental import pallas as pl
from jax.experimental.pallas import tpu as pltpu
from jax.experimental.pallas import tpu_sc as plsc

D = 768
N = 4096
NW = 32
N_PER_W = N // NW
CHUNK = 64
N_CHUNKS = N_PER_W // CHUNK
LANES = 16


def _sc_embed_add(ids, hidden, table):
    mesh = plsc.VectorSubcoreMesh(core_axis_name="c", subcore_axis_name="s")

    @functools.partial(
        pl.kernel,
        mesh=mesh,
        out_type=jax.ShapeDtypeStruct((N, D), jnp.float32),
        scratch_types=[
            pltpu.VMEM((N_PER_W,), jnp.int32),
            pltpu.VMEM((CHUNK, D), jnp.float32),
            pltpu.VMEM((CHUNK, D), jnp.float32),
            pltpu.SemaphoreType.DMA,
        ],
    )
    def k(ids_hbm, hid_hbm, table_hbm, out_hbm, idx_v, rows_v, hid_v, sem):
        wid = lax.axis_index("s") * 2 + lax.axis_index("c")
        base = wid * N_PER_W
        pltpu.sync_copy(ids_hbm.at[pl.ds(base, N_PER_W)], idx_v)
        for c in range(N_CHUNKS):
            off = base + c * CHUNK
            gather = pltpu.async_copy(
                table_hbm.at[idx_v.at[pl.ds(c * CHUNK, CHUNK)]], rows_v, sem
            )
            pltpu.sync_copy(hid_hbm.at[pl.ds(off, CHUNK)], hid_v)
            gather.wait()

            def add_row(i, carry):
                for j in range(D // LANES):
                    sl = pl.ds(j * LANES, LANES)
                    rows_v[i, sl] = rows_v[i, sl] + hid_v[i, sl]
                return carry

            lax.fori_loop(0, CHUNK, add_row, 0)
            pltpu.sync_copy(rows_v, out_hbm.at[pl.ds(off, CHUNK)])

    return k(ids, hidden, table)


def kernel(input_ids, hidden_states, positions, embed_table):
    ids = input_ids.reshape(-1).astype(jnp.int32)
    hid = hidden_states.reshape(N, D)
    out = _sc_embed_add(ids, hid, embed_table)
    return out.reshape(hidden_states.shape)


# speedup vs baseline: 1.0967x; 1.0829x over previous
import functools

import jax
import jax.numpy as jnp
from jax import lax
from jax.experimental import pallas as pl
from jax.experimental.pallas import tpu as pltpu
from jax.experimental.pallas import tpu_sc as plsc

D = 768
N = 4096
NW = 32
N_PER_W = N // NW
CHUNK = 16
N_CHUNKS = N_PER_W // CHUNK
NB = 4
LANES = 16


def _sc_embed_add(ids, hidden, table):
    mesh = plsc.VectorSubcoreMesh(core_axis_name="c", subcore_axis_name="s")

    scratch = [pltpu.VMEM((N_PER_W,), jnp.int32)]
    scratch += [pltpu.VMEM((CHUNK, D), jnp.float32) for _ in range(2 * NB)]
    scratch += [pltpu.SemaphoreType.DMA for _ in range(3 * NB)]

    @functools.partial(
        pl.kernel,
        mesh=mesh,
        out_type=jax.ShapeDtypeStruct((N, D), jnp.float32),
        scratch_types=scratch,
    )
    def k(ids_hbm, hid_hbm, table_hbm, out_hbm, idx_v, *bufs):
        rows = bufs[0:NB]
        hid = bufs[NB:2 * NB]
        gsem = bufs[2 * NB:3 * NB]
        hsem = bufs[3 * NB:4 * NB]
        osem = bufs[4 * NB:5 * NB]

        wid = lax.axis_index("s") * 2 + lax.axis_index("c")
        base = wid * N_PER_W
        pltpu.sync_copy(ids_hbm.at[pl.ds(base, N_PER_W)], idx_v)

        g = [None] * N_CHUNKS
        h = [None] * N_CHUNKS
        o = [None] * N_CHUNKS

        def issue(c):
            b = c % NB
            g[c] = pltpu.async_copy(
                table_hbm.at[idx_v.at[pl.ds(c * CHUNK, CHUNK)]], rows[b], gsem[b]
            )
            h[c] = pltpu.async_copy(
                hid_hbm.at[pl.ds(base + c * CHUNK, CHUNK)], hid[b], hsem[b]
            )

        issue(0)
        issue(1)
        for c in range(N_CHUNKS):
            b = c % NB
            if c + 2 < N_CHUNKS:
                if c - 2 >= 0:
                    o[c - 2].wait()
                issue(c + 2)
            g[c].wait()
            h[c].wait()

            def add_row(i, carry):
                for j in range(D // LANES):
                    sl = pl.ds(j * LANES, LANES)
                    rows[b][i, sl] = rows[b][i, sl] + hid[b][i, sl]
                return carry

            lax.fori_loop(0, CHUNK, add_row, 0)
            o[c] = pltpu.async_copy(
                rows[b], out_hbm.at[pl.ds(base + c * CHUNK, CHUNK)], osem[b]
            )
        for c in range(max(0, N_CHUNKS - 4), N_CHUNKS):
            o[c].wait()

    return k(ids, hidden, table)


def kernel(input_ids, hidden_states, positions, embed_table):
    ids = input_ids.reshape(-1).astype(jnp.int32)
    hid = hidden_states.reshape(N, D)
    out = _sc_embed_add(ids, hid, embed_table)
    return out.reshape(hidden_states.shape)
